# baseline (speedup 1.0000x reference)
import functools

import jax
import jax.numpy as jnp
from jax import lax
from jax.experimental import pallas as pl
from jax.experimental.pallas import tpu as pltpu
from jax.experimental.pallas import tpu_sc as plsc

B = 16
N = 8192
G = 256
M = 32
L = 16
VPN = N // L
GH = G // 2

_BIGI = 2**30


def _sc_body(x_hbm, y_hbm, z_hbm, nbh_hbm, ctr_hbm, idx_hbm,
             xv, yv, zv, dists, ccx, ccy, ccz, cd, ci,
             nbst, idxst, cstage, dbuf):
    c = lax.axis_index("c")
    s = lax.axis_index("s")
    b = c * 8 + s // 2
    h = s % 2
    g0 = h * GH
    iota = jnp.arange(L, dtype=jnp.int32)
    lane0 = iota == 0
    inf_v = jnp.full((L,), jnp.inf, dtype=jnp.float32)

    pltpu.sync_copy(x_hbm.at[b], xv)
    pltpu.sync_copy(y_hbm.at[b], yv)
    pltpu.sync_copy(z_hbm.at[b], zv)

    def init_d(j, carry):
        dists[pl.ds(j * L, L)] = inf_v
        return carry

    lax.fori_loop(0, VPN, init_d, 0)

    def fps_step(g, widx):
        wv = jnp.full((L,), widx, dtype=jnp.int32)
        cx = plsc.load_gather(xv, [wv])
        cy = plsc.load_gather(yv, [wv])
        cz = plsc.load_gather(zv, [wv])
        gv = jnp.full((L,), g, dtype=jnp.int32)
        plsc.store_scatter(ccx, [gv], cx, mask=lane0)
        plsc.store_scatter(ccy, [gv], cy, mask=lane0)
        plsc.store_scatter(ccz, [gv], cz, mask=lane0)

        bv0 = jnp.full((L,), -jnp.inf, dtype=jnp.float32)
        bi0 = jnp.zeros((L,), dtype=jnp.int32)

        @plsc.parallel_loop(0, VPN, carry=(bv0, bi0), unroll=8)
        def scan(j, st):
            bv, bi = st
            off = j * L
            xj = xv[pl.ds(off, L)]
            yj = yv[pl.ds(off, L)]
            zj = zv[pl.ds(off, L)]
            dj = dists[pl.ds(off, L)]
            dx = xj - cx
            dy = yj - cy
            dz = zj - cz
            d = dx * dx + dy * dy + dz * dz
            dn = jnp.minimum(dj, d)
            dists[pl.ds(off, L)] = dn
            upd = dn > bv
            gidx = jnp.full((L,), off, dtype=jnp.int32) + iota
            bi = jnp.where(upd, gidx, bi)
            bv = jnp.where(upd, dn, bv)
            return bv, bi

        bv, bi = scan
        m = jnp.max(bv)
        mv = jnp.full((L,), m, dtype=jnp.float32)
        cand = jnp.where(bv == mv, bi, jnp.full((L,), _BIGI, dtype=jnp.int32))
        return jnp.min(cand)

    lax.fori_loop(0, G, fps_step, jnp.int32(0))

    C = 4

    def insert_into(k, dref_base, qual, off):
        kb = k * M
        offv = jnp.full((L,), off, dtype=jnp.int32)
        dbv = jnp.full((L,), dref_base, dtype=jnp.int32) + offv

        def w_body(mask):
            lanev = plsc.all_reduce_ffs(mask)
            ddv = plsc.load_gather(dbuf, [dbv + lanev])
            iiv = offv + lanev
            cdA = cd[pl.ds(kb, L)]
            cdB = cd[pl.ds(kb + L, L)]
            ciA = ci[pl.ds(kb, L)]
            ciB = ci[pl.ds(kb + L, L)]
            cntA = plsc.all_reduce_population_count(cdA <= ddv)
            cntB = plsc.all_reduce_population_count(cdB <= ddv)
            posv = cntA + cntB
            npA = iota + jnp.where(iota >= posv, 1, 0).astype(jnp.int32)
            rkB = iota + L
            npB = rkB + jnp.where(rkB >= posv, 1, 0).astype(jnp.int32)
            mB = npB < M
            kbv = jnp.full((L,), kb, dtype=jnp.int32)
            plsc.store_scatter(cd, [kbv + npA], cdA)
            plsc.store_scatter(ci, [kbv + npA], ciA)
            plsc.store_scatter(cd, [kbv + npB], cdB, mask=mB)
            plsc.store_scatter(ci, [kbv + npB], ciB, mask=mB)
            insm = lane0 & (posv < M)
            plsc.store_scatter(cd, [kbv + posv], ddv, mask=insm)
            plsc.store_scatter(ci, [kbv + posv], iiv, mask=insm)
            return mask & (iota != lanev)

        lax.while_loop(lambda mk: jnp.any(mk), w_body, qual)
        t = jnp.max(cd[pl.ds(kb + L, L)])
        return jnp.full((L,), t, dtype=jnp.float32)

    def knn_quad(r, carry):
        gbase = g0 + r * C
        cs = []
        for k in range(C):
            gv = jnp.full((L,), gbase + k, dtype=jnp.int32)
            cs.append((plsc.load_gather(ccx, [gv]),
                       plsc.load_gather(ccy, [gv]),
                       plsc.load_gather(ccz, [gv])))
            cd[pl.ds(k * M, L)] = inf_v
            cd[pl.ds(k * M + L, L)] = inf_v
            ci[pl.ds(k * M, L)] = jnp.full((L,), _BIGI, dtype=jnp.int32)
            ci[pl.ds(k * M + L, L)] = jnp.full((L,), _BIGI, dtype=jnp.int32)

        @plsc.parallel_loop(0, VPN, unroll=4)
        def dpass(j):
            off = j * L
            xj = xv[pl.ds(off, L)]
            yj = yv[pl.ds(off, L)]
            zj = zv[pl.ds(off, L)]
            for k in range(C):
                cx, cy, cz = cs[k]
                dx = xj - cx
                dy = yj - cy
                dz = zj - cz
                d = dx * dx + dy * dy + dz * dz
                dbuf[pl.ds(k * N + off, L)] = d

        for k in range(C):
            def stream(j, tauv, k=k):
                off = j * L
                d = dbuf[pl.ds(k * N + off, L)]
                qual = d < tauv

                def insert_all(k=k, qual=qual, off=off):
                    return insert_into(k, k * N, qual, off)

                return lax.cond(jnp.any(qual), insert_all, lambda: tauv)

            lax.fori_loop(0, VPN, stream, inf_v)

        for k in range(C):
            row = r * C + k
            ciA = ci[pl.ds(k * M, L)]
            ciB = ci[pl.ds(k * M + L, L)]
            idxst[pl.ds(row * M, L)] = ciA
            idxst[pl.ds(row * M + L, L)] = ciB
            pA = jnp.full((L,), row * M * 3, dtype=jnp.int32) + iota * 3
            pB = pA + L * 3
            for civec, pbase in ((ciA, pA), (ciB, pB)):
                xg = plsc.load_gather(xv, [civec])
                yg = plsc.load_gather(yv, [civec])
                zg = plsc.load_gather(zv, [civec])
                plsc.store_scatter(nbst, [pbase], xg)
                plsc.store_scatter(nbst, [pbase + 1], yg)
                plsc.store_scatter(nbst, [pbase + 2], zg)
        return carry

    lax.fori_loop(0, GH // C, knn_quad, 0)

    nb_off = (b * G + g0) * (M * 3)
    pltpu.sync_copy(nbst, nbh_hbm.at[pl.ds(nb_off, GH * M * 3)])
    ix_off = (b * G + g0) * M
    pltpu.sync_copy(idxst, idx_hbm.at[pl.ds(ix_off, GH * M)])

    @pl.when(h == 0)
    def _():
        def cw(t, carry):
            cxw = ccx[pl.ds(t * L, L)]
            cyw = ccy[pl.ds(t * L, L)]
            czw = ccz[pl.ds(t * L, L)]
            pbase = (jnp.full((L,), t * L, dtype=jnp.int32) + iota) * 3
            plsc.store_scatter(cstage, [pbase], cxw)
            plsc.store_scatter(cstage, [pbase + 1], cyw)
            plsc.store_scatter(cstage, [pbase + 2], czw)
            return carry

        lax.fori_loop(0, G // L, cw, 0)
        pltpu.sync_copy(cstage, ctr_hbm.at[pl.ds(b * G * 3, G * 3)])


@functools.partial(jax.jit, static_argnums=())
def _sc_group(x, y, z):
    mesh = plsc.VectorSubcoreMesh(
        core_axis_name="c", subcore_axis_name="s", num_cores=2, num_subcores=16)
    kern = pl.kernel(
        _sc_body,
        out_type=(
            jax.ShapeDtypeStruct((B * G * M * 3,), jnp.float32),
            jax.ShapeDtypeStruct((B * G * 3,), jnp.float32),
            jax.ShapeDtypeStruct((B * G * M,), jnp.int32),
        ),
        mesh=mesh,
        compiler_params=pltpu.CompilerParams(needs_layout_passes=False),
        scratch_types=[
            pltpu.VMEM((N,), jnp.float32),
            pltpu.VMEM((N,), jnp.float32),
            pltpu.VMEM((N,), jnp.float32),
            pltpu.VMEM((N,), jnp.float32),
            pltpu.VMEM((G,), jnp.float32),
            pltpu.VMEM((G,), jnp.float32),
            pltpu.VMEM((G,), jnp.float32),
            pltpu.VMEM((4 * M,), jnp.float32),
            pltpu.VMEM((4 * M,), jnp.int32),
            pltpu.VMEM((GH * M * 3,), jnp.float32),
            pltpu.VMEM((GH * M,), jnp.int32),
            pltpu.VMEM((G * 3,), jnp.float32),
            pltpu.VMEM((4 * N,), jnp.float32),
        ],
    )
    return kern(x, y, z)


def kernel(xyz):
    xt = jnp.transpose(xyz, (2, 0, 1))
    x, y, z = xt[0], xt[1], xt[2]
    nbf, cf, ixf = _sc_group(x, y, z)
    neighborhood = nbf.reshape(B, G, M, 3)
    center = cf.reshape(B, G, 3)
    idx_group = ixf.reshape(B, G, M)
    return (neighborhood, center, idx_group)


# speedup vs baseline: 1.2482x; 1.2482x over previous
import functools

import jax
import jax.numpy as jnp
from jax import lax
from jax.experimental import pallas as pl
from jax.experimental.pallas import tpu as pltpu
from jax.experimental.pallas import tpu_sc as plsc

B = 16
N = 8192
G = 256
M = 32
L = 16
VPN = N // L
GH = G // 2

_BIGI = 2**30


def _sc_body(x_hbm, y_hbm, z_hbm, nbh_hbm, ctr_hbm, idx_hbm,
             xv, yv, zv, dists, ccx, ccy, ccz, cd, ci,
             nbst, idxst, cstage, sv_st):
    c = lax.axis_index("c")
    s = lax.axis_index("s")
    b = c * 8 + s // 2
    h = s % 2
    g0 = h * GH
    iota = jnp.arange(L, dtype=jnp.int32)
    lane0 = iota == 0
    inf_v = jnp.full((L,), jnp.inf, dtype=jnp.float32)

    pltpu.sync_copy(x_hbm.at[b], xv)
    pltpu.sync_copy(y_hbm.at[b], yv)
    pltpu.sync_copy(z_hbm.at[b], zv)

    def init_d(j, carry):
        dists[pl.ds(j * L, L)] = inf_v
        return carry

    lax.fori_loop(0, VPN, init_d, 0)

    def fps_step(g, widx):
        wv = jnp.full((L,), widx, dtype=jnp.int32)
        cx = plsc.load_gather(xv, [wv])
        cy = plsc.load_gather(yv, [wv])
        cz = plsc.load_gather(zv, [wv])
        gv = jnp.full((L,), g, dtype=jnp.int32)
        plsc.store_scatter(ccx, [gv], cx, mask=lane0)
        plsc.store_scatter(ccy, [gv], cy, mask=lane0)
        plsc.store_scatter(ccz, [gv], cz, mask=lane0)

        bv0 = jnp.full((L,), -jnp.inf, dtype=jnp.float32)
        bi0 = jnp.zeros((L,), dtype=jnp.int32)

        @plsc.parallel_loop(0, VPN, carry=(bv0, bi0), unroll=8)
        def scan(j, st):
            bv, bi = st
            off = j * L
            xj = xv[pl.ds(off, L)]
            yj = yv[pl.ds(off, L)]
            zj = zv[pl.ds(off, L)]
            dj = dists[pl.ds(off, L)]
            dx = xj - cx
            dy = yj - cy
            dz = zj - cz
            d = dx * dx + dy * dy + dz * dz
            dn = jnp.minimum(dj, d)
            dists[pl.ds(off, L)] = dn
            upd = dn > bv
            gidx = jnp.full((L,), off, dtype=jnp.int32) + iota
            bi = jnp.where(upd, gidx, bi)
            bv = jnp.where(upd, dn, bv)
            return bv, bi

        bv, bi = scan
        m = jnp.max(bv)
        mv = jnp.full((L,), m, dtype=jnp.float32)
        cand = jnp.where(bv == mv, bi, jnp.full((L,), _BIGI, dtype=jnp.int32))
        return jnp.min(cand)

    lax.fori_loop(0, G, fps_step, jnp.int32(0))

    C = 8

    def insert_into(k, d, qual, off):
        kb = k * M
        sv_st[...] = d
        offv = jnp.full((L,), off, dtype=jnp.int32)

        def w_body(mask):
            lanev = plsc.all_reduce_ffs(mask)
            ddv = plsc.load_gather(sv_st, [lanev])
            iiv = offv + lanev
            cdA = cd[pl.ds(kb, L)]
            cdB = cd[pl.ds(kb + L, L)]
            ciA = ci[pl.ds(kb, L)]
            ciB = ci[pl.ds(kb + L, L)]
            cntA = plsc.all_reduce_population_count(cdA <= ddv)
            cntB = plsc.all_reduce_population_count(cdB <= ddv)
            posv = cntA + cntB
            npA = iota + jnp.where(iota >= posv, 1, 0).astype(jnp.int32)
            rkB = iota + L
            npB = rkB + jnp.where(rkB >= posv, 1, 0).astype(jnp.int32)
            mB = npB < M
            kbv = jnp.full((L,), kb, dtype=jnp.int32)
            plsc.store_scatter(cd, [kbv + npA], cdA)
            plsc.store_scatter(ci, [kbv + npA], ciA)
            plsc.store_scatter(cd, [kbv + npB], cdB, mask=mB)
            plsc.store_scatter(ci, [kbv + npB], ciB, mask=mB)
            insm = lane0 & (posv < M)
            plsc.store_scatter(cd, [kbv + posv], ddv, mask=insm)
            plsc.store_scatter(ci, [kbv + posv], iiv, mask=insm)
            return mask & (iota != lanev)

        lax.while_loop(lambda mk: jnp.any(mk), w_body, qual)
        t = jnp.max(cd[pl.ds(kb + L, L)])
        return jnp.full((L,), t, dtype=jnp.float32)

    def knn_quad(r, carry):
        gbase = g0 + r * C
        cs = []
        for k in range(C):
            gv = jnp.full((L,), gbase + k, dtype=jnp.int32)
            cs.append((plsc.load_gather(ccx, [gv]),
                       plsc.load_gather(ccy, [gv]),
                       plsc.load_gather(ccz, [gv])))
            cd[pl.ds(k * M, L)] = inf_v
            cd[pl.ds(k * M + L, L)] = inf_v
            ci[pl.ds(k * M, L)] = jnp.full((L,), _BIGI, dtype=jnp.int32)
            ci[pl.ds(k * M + L, L)] = jnp.full((L,), _BIGI, dtype=jnp.int32)

        def stream(j, taus):
            off = j * L
            xj = xv[pl.ds(off, L)]
            yj = yv[pl.ds(off, L)]
            zj = zv[pl.ds(off, L)]
            ds_ = []
            quals = []
            for k in range(C):
                cx, cy, cz = cs[k]
                dx = xj - cx
                dy = yj - cy
                dz = zj - cz
                d = dx * dx + dy * dy + dz * dz
                ds_.append(d)
                quals.append(d < taus[k])
            anyq = ((quals[0] | quals[1]) | (quals[2] | quals[3])) | \
                   ((quals[4] | quals[5]) | (quals[6] | quals[7]))

            def insert_all():
                new_taus = []
                for k in range(C):
                    new_taus.append(lax.cond(
                        jnp.any(quals[k]),
                        functools.partial(insert_into, k, ds_[k], quals[k], off),
                        lambda tk=taus[k]: tk))
                return tuple(new_taus)

            return lax.cond(jnp.any(anyq), insert_all, lambda: taus)

        lax.fori_loop(0, VPN, stream, (inf_v,) * C)

        for k in range(C):
            row = r * C + k
            ciA = ci[pl.ds(k * M, L)]
            ciB = ci[pl.ds(k * M + L, L)]
            idxst[pl.ds(row * M, L)] = ciA
            idxst[pl.ds(row * M + L, L)] = ciB
            pA = jnp.full((L,), row * M * 3, dtype=jnp.int32) + iota * 3
            pB = pA + L * 3
            for civec, pbase in ((ciA, pA), (ciB, pB)):
                xg = plsc.load_gather(xv, [civec])
                yg = plsc.load_gather(yv, [civec])
                zg = plsc.load_gather(zv, [civec])
                plsc.store_scatter(nbst, [pbase], xg)
                plsc.store_scatter(nbst, [pbase + 1], yg)
                plsc.store_scatter(nbst, [pbase + 2], zg)
        return carry

    lax.fori_loop(0, GH // C, knn_quad, 0)

    nb_off = (b * G + g0) * (M * 3)
    pltpu.sync_copy(nbst, nbh_hbm.at[pl.ds(nb_off, GH * M * 3)])
    ix_off = (b * G + g0) * M
    pltpu.sync_copy(idxst, idx_hbm.at[pl.ds(ix_off, GH * M)])

    @pl.when(h == 0)
    def _():
        def cw(t, carry):
            cxw = ccx[pl.ds(t * L, L)]
            cyw = ccy[pl.ds(t * L, L)]
            czw = ccz[pl.ds(t * L, L)]
            pbase = (jnp.full((L,), t * L, dtype=jnp.int32) + iota) * 3
            plsc.store_scatter(cstage, [pbase], cxw)
            plsc.store_scatter(cstage, [pbase + 1], cyw)
            plsc.store_scatter(cstage, [pbase + 2], czw)
            return carry

        lax.fori_loop(0, G // L, cw, 0)
        pltpu.sync_copy(cstage, ctr_hbm.at[pl.ds(b * G * 3, G * 3)])


@functools.partial(jax.jit, static_argnums=())
def _sc_group(x, y, z):
    mesh = plsc.VectorSubcoreMesh(
        core_axis_name="c", subcore_axis_name="s", num_cores=2, num_subcores=16)
    kern = pl.kernel(
        _sc_body,
        out_type=(
            jax.ShapeDtypeStruct((B * G * M * 3,), jnp.float32),
            jax.ShapeDtypeStruct((B * G * 3,), jnp.float32),
            jax.ShapeDtypeStruct((B * G * M,), jnp.int32),
        ),
        mesh=mesh,
        compiler_params=pltpu.CompilerParams(needs_layout_passes=False),
        scratch_types=[
            pltpu.VMEM((N,), jnp.float32),
            pltpu.VMEM((N,), jnp.float32),
            pltpu.VMEM((N,), jnp.float32),
            pltpu.VMEM((N,), jnp.float32),
            pltpu.VMEM((G,), jnp.float32),
            pltpu.VMEM((G,), jnp.float32),
            pltpu.VMEM((G,), jnp.float32),
            pltpu.VMEM((8 * M,), jnp.float32),
            pltpu.VMEM((8 * M,), jnp.int32),
            pltpu.VMEM((GH * M * 3,), jnp.float32),
            pltpu.VMEM((GH * M,), jnp.int32),
            pltpu.VMEM((G * 3,), jnp.float32),
            pltpu.VMEM((L,), jnp.float32),
        ],
    )
    return kern(x, y, z)


def kernel(xyz):
    xt = jnp.transpose(xyz, (2, 0, 1))
    x, y, z = xt[0], xt[1], xt[2]
    nbf, cf, ixf = _sc_group(x, y, z)
    neighborhood = nbf.reshape(B, G, M, 3)
    center = cf.reshape(B, G, 3)
    idx_group = ixf.reshape(B, G, M)
    return (neighborhood, center, idx_group)


# speedup vs baseline: 1.3355x; 1.0699x over previous
import functools

import jax
import jax.numpy as jnp
from jax import lax
from jax.experimental import pallas as pl
from jax.experimental.pallas import tpu as pltpu
from jax.experimental.pallas import tpu_sc as plsc

B = 16
N = 8192
G = 256
M = 32
L = 16
VPN = N // L
NHALF = N // 2
VPH = NHALF // L
GH = G // 2

_BIGI = 2**30


def _sc_body(x_hbm, y_hbm, z_hbm, nbh_hbm, ctr_hbm, idx_hbm,
             xv, yv, zv, dists, ccx, ccy, ccz, cd, ci,
             nbst, idxst, cstage, sv_st, recv_sm):
    c = lax.axis_index("c")
    s = lax.axis_index("s")
    b = c * 8 + s // 2
    h = s % 2
    base = h * NHALF
    g0 = h * GH
    iota = jnp.arange(L, dtype=jnp.int32)
    lane0 = iota == 0
    inf_v = jnp.full((L,), jnp.inf, dtype=jnp.float32)

    recv_sm[0] = jnp.int32(0)
    recv_sm[1] = jnp.int32(0)
    recv_sm[2] = jnp.int32(0)
    recv_sm[3] = jnp.int32(0)

    pltpu.sync_copy(x_hbm.at[b], xv)
    pltpu.sync_copy(y_hbm.at[b], yv)
    pltpu.sync_copy(z_hbm.at[b], zv)

    def init_d(j, carry):
        dists[pl.ds(j * L, L)] = inf_v
        return carry

    lax.fori_loop(0, VPH, init_d, 0)
    plsc.subcore_barrier()

    def fps_step(g, widx):
        wv = jnp.full((L,), widx, dtype=jnp.int32)
        cx = plsc.load_gather(xv, [wv])
        cy = plsc.load_gather(yv, [wv])
        cz = plsc.load_gather(zv, [wv])
        gv = jnp.full((L,), g, dtype=jnp.int32)
        plsc.store_scatter(ccx, [gv], cx, mask=lane0)
        plsc.store_scatter(ccy, [gv], cy, mask=lane0)
        plsc.store_scatter(ccz, [gv], cz, mask=lane0)

        bv0 = jnp.full((L,), -jnp.inf, dtype=jnp.float32)
        bi0 = jnp.zeros((L,), dtype=jnp.int32)

        @plsc.parallel_loop(0, VPH, carry=(bv0, bi0), unroll=8)
        def scan(j, st):
            bv, bi = st
            off = j * L
            xj = xv[pl.ds(base + off, L)]
            yj = yv[pl.ds(base + off, L)]
            zj = zv[pl.ds(base + off, L)]
            dj = dists[pl.ds(off, L)]
            dx = xj - cx
            dy = yj - cy
            dz = zj - cz
            d = dx * dx + dy * dy + dz * dz
            dn = jnp.minimum(dj, d)
            dists[pl.ds(off, L)] = dn
            upd = dn > bv
            gidx = jnp.full((L,), base + off, dtype=jnp.int32) + iota
            bi = jnp.where(upd, gidx, bi)
            bv = jnp.where(upd, dn, bv)
            return bv, bi

        bv, bi = scan
        m = jnp.max(bv)
        mv = jnp.full((L,), m, dtype=jnp.float32)
        cand = jnp.where(bv == mv, bi, jnp.full((L,), _BIGI, dtype=jnp.int32))
        li = jnp.min(cand)
        mb = jnp.max(plsc.bitcast(mv, jnp.int32)) + 1
        p = g % 2
        plsc.fetch_and_add(recv_sm.at[2 * p], mb, subcore_id=s ^ 1)
        plsc.fetch_and_add(recv_sm.at[2 * p + 1], li + 1, subcore_id=s ^ 1)

        def spin_cond(st2):
            pvb, pib = st2
            return (pvb == 0) | (pib == 0)

        def spin_body(st2):
            return recv_sm[2 * p], recv_sm[2 * p + 1]

        pvb, pib = lax.while_loop(spin_cond, spin_body,
                                  (recv_sm[2 * p], recv_sm[2 * p + 1]))
        recv_sm[2 * p] = jnp.int32(0)
        recv_sm[2 * p + 1] = jnp.int32(0)
        pi = pib - 1
        take_p = (pvb > mb) | ((pvb == mb) & (pi < li))
        return jnp.where(take_p, pi, li)

    lax.fori_loop(0, G, fps_step, jnp.int32(0))

    C = 8

    def insert_into(k, d, qual, off):
        kb = k * M
        sv_st[...] = d
        offv = jnp.full((L,), off, dtype=jnp.int32)

        def w_body(mask):
            lanev = plsc.all_reduce_ffs(mask)
            ddv = plsc.load_gather(sv_st, [lanev])
            iiv = offv + lanev
            cdA = cd[pl.ds(kb, L)]
            cdB = cd[pl.ds(kb + L, L)]
            ciA = ci[pl.ds(kb, L)]
            ciB = ci[pl.ds(kb + L, L)]
            cntA = plsc.all_reduce_population_count(cdA <= ddv)
            cntB = plsc.all_reduce_population_count(cdB <= ddv)
            posv = cntA + cntB
            npA = iota + jnp.where(iota >= posv, 1, 0).astype(jnp.int32)
            rkB = iota + L
            npB = rkB + jnp.where(rkB >= posv, 1, 0).astype(jnp.int32)
            mB = npB < M
            kbv = jnp.full((L,), kb, dtype=jnp.int32)
            plsc.store_scatter(cd, [kbv + npA], cdA)
            plsc.store_scatter(ci, [kbv + npA], ciA)
            plsc.store_scatter(cd, [kbv + npB], cdB, mask=mB)
            plsc.store_scatter(ci, [kbv + npB], ciB, mask=mB)
            insm = lane0 & (posv < M)
            plsc.store_scatter(cd, [kbv + posv], ddv, mask=insm)
            plsc.store_scatter(ci, [kbv + posv], iiv, mask=insm)
            return mask & (iota != lanev)

        lax.while_loop(lambda mk: jnp.any(mk), w_body, qual)
        t = jnp.max(cd[pl.ds(kb + L, L)])
        return jnp.full((L,), t, dtype=jnp.float32)

    def knn_quad(r, carry):
        gbase = g0 + r * C
        cs = []
        for k in range(C):
            gv = jnp.full((L,), gbase + k, dtype=jnp.int32)
            cs.append((plsc.load_gather(ccx, [gv]),
                       plsc.load_gather(ccy, [gv]),
                       plsc.load_gather(ccz, [gv])))
            cd[pl.ds(k * M, L)] = inf_v
            cd[pl.ds(k * M + L, L)] = inf_v
            ci[pl.ds(k * M, L)] = jnp.full((L,), _BIGI, dtype=jnp.int32)
            ci[pl.ds(k * M + L, L)] = jnp.full((L,), _BIGI, dtype=jnp.int32)

        def stream(j, taus):
            off = j * L
            xj = xv[pl.ds(off, L)]
            yj = yv[pl.ds(off, L)]
            zj = zv[pl.ds(off, L)]
            ds_ = []
            quals = []
            for k in range(C):
                cx, cy, cz = cs[k]
                dx = xj - cx
                dy = yj - cy
                dz = zj - cz
                d = dx * dx + dy * dy + dz * dz
                ds_.append(d)
                quals.append(d < taus[k])
            anyq = ((quals[0] | quals[1]) | (quals[2] | quals[3])) | \
                   ((quals[4] | quals[5]) | (quals[6] | quals[7]))

            def insert_all():
                new_taus = []
                for k in range(C):
                    new_taus.append(lax.cond(
                        jnp.any(quals[k]),
                        functools.partial(insert_into, k, ds_[k], quals[k], off),
                        lambda tk=taus[k]: tk))
                return tuple(new_taus)

            return lax.cond(jnp.any(anyq), insert_all, lambda: taus)

        lax.fori_loop(0, VPN, stream, (inf_v,) * C)

        for k in range(C):
            row = r * C + k
            ciA = ci[pl.ds(k * M, L)]
            ciB = ci[pl.ds(k * M + L, L)]
            idxst[pl.ds(row * M, L)] = ciA
            idxst[pl.ds(row * M + L, L)] = ciB
            pA = jnp.full((L,), row * M * 3, dtype=jnp.int32) + iota * 3
            pB = pA + L * 3
            for civec, pbase in ((ciA, pA), (ciB, pB)):
                xg = plsc.load_gather(xv, [civec])
                yg = plsc.load_gather(yv, [civec])
                zg = plsc.load_gather(zv, [civec])
                plsc.store_scatter(nbst, [pbase], xg)
                plsc.store_scatter(nbst, [pbase + 1], yg)
                plsc.store_scatter(nbst, [pbase + 2], zg)
        return carry

    lax.fori_loop(0, GH // C, knn_quad, 0)

    nb_off = (b * G + g0) * (M * 3)
    pltpu.sync_copy(nbst, nbh_hbm.at[pl.ds(nb_off, GH * M * 3)])
    ix_off = (b * G + g0) * M
    pltpu.sync_copy(idxst, idx_hbm.at[pl.ds(ix_off, GH * M)])

    @pl.when(h == 0)
    def _():
        def cw(t, carry):
            cxw = ccx[pl.ds(t * L, L)]
            cyw = ccy[pl.ds(t * L, L)]
            czw = ccz[pl.ds(t * L, L)]
            pbase = (jnp.full((L,), t * L, dtype=jnp.int32) + iota) * 3
            plsc.store_scatter(cstage, [pbase], cxw)
            plsc.store_scatter(cstage, [pbase + 1], cyw)
            plsc.store_scatter(cstage, [pbase + 2], czw)
            return carry

        lax.fori_loop(0, G // L, cw, 0)
        pltpu.sync_copy(cstage, ctr_hbm.at[pl.ds(b * G * 3, G * 3)])


@functools.partial(jax.jit, static_argnums=())
def _sc_group(x, y, z):
    mesh = plsc.VectorSubcoreMesh(
        core_axis_name="c", subcore_axis_name="s", num_cores=2, num_subcores=16)
    kern = pl.kernel(
        _sc_body,
        out_type=(
            jax.ShapeDtypeStruct((B * G * M * 3,), jnp.float32),
            jax.ShapeDtypeStruct((B * G * 3,), jnp.float32),
            jax.ShapeDtypeStruct((B * G * M,), jnp.int32),
        ),
        mesh=mesh,
        compiler_params=pltpu.CompilerParams(needs_layout_passes=False),
        scratch_types=[
            pltpu.VMEM((N,), jnp.float32),
            pltpu.VMEM((N,), jnp.float32),
            pltpu.VMEM((N,), jnp.float32),
            pltpu.VMEM((NHALF,), jnp.float32),
            pltpu.VMEM((G,), jnp.float32),
            pltpu.VMEM((G,), jnp.float32),
            pltpu.VMEM((G,), jnp.float32),
            pltpu.VMEM((8 * M,), jnp.float32),
            pltpu.VMEM((8 * M,), jnp.int32),
            pltpu.VMEM((GH * M * 3,), jnp.float32),
            pltpu.VMEM((GH * M,), jnp.int32),
            pltpu.VMEM((G * 3,), jnp.float32),
            pltpu.VMEM((L,), jnp.float32),
            pltpu.SMEM((4,), jnp.int32),
        ],
    )
    return kern(x, y, z)


def kernel(xyz):
    xt = jnp.transpose(xyz, (2, 0, 1))
    x, y, z = xt[0], xt[1], xt[2]
    nbf, cf, ixf = _sc_group(x, y, z)
    neighborhood = nbf.reshape(B, G, M, 3)
    center = cf.reshape(B, G, 3)
    idx_group = ixf.reshape(B, G, M)
    return (neighborhood, center, idx_group)
